# Initial kernel scaffold; baseline (speedup 1.0000x reference)
import jax
import jax.numpy as jnp
from jax.experimental import pallas as pl


def kernel(x, edge_index, edge_attr, g1_Ws, g1_bs, g1_We, g1_be, g1_W1, g1_b1, g1_W2, g1_b2, g2_Ws, g2_bs, g2_We, g2_be, g2_W1, g2_b1, g2_W2, g2_b2, mu_W, mu_b, ls_W, ls_b):
    raise NotImplementedError("write your pallas kernel here")


# speedup vs baseline: 7.4295x; 7.4295x over previous
import functools
import math

import jax
import jax.numpy as jnp
from jax import lax
from jax.experimental import pallas as pl
from jax.experimental.pallas import tpu as pltpu
from jax.experimental.pallas import tpu_sc as plsc

EPS = 1e-7
NPAD = 10016
CHUNK = 80
NCORES = 2
NSUB = 16
NW = NCORES * NSUB
ROWS_PER_SUB = NPAD // NSUB
ZFULL = ROWS_PER_SUB // CHUNK
ZREM = ROWS_PER_SUB % CHUNK


def _dense_body(bounds, x_ref, w_ref, b_ref, *refs):
    ns = len(bounds) - 1
    i = pl.program_id(0)
    o = jnp.dot(x_ref[...], w_ref[...], preferred_element_type=jnp.float32)
    o = o + b_ref[...]
    for s in range(ns):
        lo, hi = bounds[s], bounds[s + 1]
        blkc = o[:, lo:hi]
        refs[s][...] = blkc
        bm = jnp.max(blkc, axis=0, keepdims=True)
        m_ref = refs[ns + s]

        @pl.when(i == 0)
        def _():
            m_ref[...] = bm

        @pl.when(i != 0)
        def _():
            m_ref[...] = jnp.maximum(m_ref[...], bm)


def _dense(x, w, b, blk, splits=None):
    n, k = x.shape
    co = w.shape[1]
    if splits is None:
        splits = (co,)
    bounds = [0]
    for s in splits:
        bounds.append(bounds[-1] + s)
    out_shape = ([jax.ShapeDtypeStruct((n, s), jnp.float32) for s in splits]
                 + [jax.ShapeDtypeStruct((1, s), jnp.float32) for s in splits])
    out_specs = ([pl.BlockSpec((blk, s), lambda i: (i, 0)) for s in splits]
                 + [pl.BlockSpec((1, s), lambda i: (0, 0)) for s in splits])
    return pl.pallas_call(
        functools.partial(_dense_body, tuple(bounds)),
        grid=(n // blk,),
        in_specs=[pl.BlockSpec((blk, k), lambda i: (i, 0)),
                  pl.BlockSpec((k, co), lambda i: (0, 0)),
                  pl.BlockSpec((1, co), lambda i: (0, 0))],
        out_specs=out_specs,
        out_shape=out_shape,
    )(x, w, b.reshape(1, co))


def _mlp_body(nh, w1_ref, b1_ref, w2_ref, b2_ref, *refs, o_ref):
    parts = []
    for s in range(nh):
        a0_ref, a1_ref, h_ref = refs[3 * s], refs[3 * s + 1], refs[3 * s + 2]
        ch = h_ref.shape[1]
        sm = a0_ref[:, :ch] + a1_ref[:, :ch]
        wm = a0_ref[:, ch:] + a1_ref[:, ch:]
        parts.append(wm / (sm + 1e-16) + h_ref[...])
    out = parts[0] if nh == 1 else jnp.concatenate(parts, axis=1)
    t = jnp.dot(out, w1_ref[...], preferred_element_type=jnp.float32) + b1_ref[...]
    t = jnp.maximum(t, 0.0)
    t = jnp.dot(t, w2_ref[...], preferred_element_type=jnp.float32) + b2_ref[...]
    o_ref[...] = jnp.maximum(t, 0.0)


def _mlp_wrap(nh, *args):
    o_ref = args[-1]
    w1_ref, b1_ref, w2_ref, b2_ref = args[:4]
    _mlp_body(nh, w1_ref, b1_ref, w2_ref, b2_ref, *args[4:-1], o_ref=o_ref)


def _mlp(pairs, w1, b1, w2, b2, blk=2000):
    co = w1.shape[0]
    ch_mlp = w1.shape[1]
    n = pairs[0][1].shape[0]
    ins = [w1, b1.reshape(1, ch_mlp), w2, b2.reshape(1, co)]
    in_specs = [pl.BlockSpec((co, ch_mlp), lambda i: (0, 0)),
                pl.BlockSpec((1, ch_mlp), lambda i: (0, 0)),
                pl.BlockSpec((ch_mlp, co), lambda i: (0, 0)),
                pl.BlockSpec((1, co), lambda i: (0, 0))]
    for acc, h in pairs:
        ch2 = acc.shape[1]
        ch = h.shape[1]
        ins += [acc[:NPAD], acc[NPAD:], h]
        in_specs += [pl.BlockSpec((blk, ch2), lambda i: (i, 0)),
                     pl.BlockSpec((blk, ch2), lambda i: (i, 0)),
                     pl.BlockSpec((blk, ch), lambda i: (i, 0))]
    return pl.pallas_call(
        functools.partial(_mlp_wrap, len(pairs)),
        grid=(n // blk,),
        in_specs=in_specs,
        out_specs=pl.BlockSpec((blk, co), lambda i: (i, 0)),
        out_shape=jax.ShapeDtypeStruct((n, co), jnp.float32),
    )(*ins)


def _gcnpre_body(x_ref, w_ref, d0_ref, d1_ref, tab_ref, slf_ref, dv_ref):
    deg = 1.0 + d0_ref[:, 0:1] + d1_ref[:, 0:1]
    dinv = lax.rsqrt(deg)
    hc = jnp.dot(x_ref[...], w_ref[...], preferred_element_type=jnp.float32)
    tab_ref[...] = hc * dinv
    slf_ref[...] = hc * (dinv * dinv)
    dv_ref[...] = dinv


def _gcn_pre(x, wcat, dega, blk=2000):
    n, k = x.shape
    co2 = wcat.shape[1]
    d0 = dega[:NPAD]
    d1 = dega[NPAD:]
    return pl.pallas_call(
        _gcnpre_body,
        grid=(n // blk,),
        in_specs=[pl.BlockSpec((blk, k), lambda i: (i, 0)),
                  pl.BlockSpec((k, co2), lambda i: (0, 0)),
                  pl.BlockSpec((blk, 16), lambda i: (i, 0)),
                  pl.BlockSpec((blk, 16), lambda i: (i, 0))],
        out_specs=[pl.BlockSpec((blk, co2), lambda i: (i, 0)),
                   pl.BlockSpec((blk, co2), lambda i: (i, 0)),
                   pl.BlockSpec((blk, 1), lambda i: (i, 0))],
        out_shape=[jax.ShapeDtypeStruct((n, co2), jnp.float32),
                   jax.ShapeDtypeStruct((n, co2), jnp.float32),
                   jax.ShapeDtypeStruct((n, 1), jnp.float32)],
    )(x, wcat, d0, d1)


def _gcnpost_body(co, g0_ref, g1_ref, slf_ref, dv_ref, bm_ref, bl_ref, mu_ref, ls_ref):
    agg = g0_ref[...] + g1_ref[...]
    out = dv_ref[...] * agg + slf_ref[...]
    mu_ref[...] = out[:, :co] + bm_ref[...]
    ls_ref[...] = out[:, co:2 * co] + bl_ref[...]


def _gcn_post(gacc, slf, dv, mu_b, ls_b, blk=2000):
    n = slf.shape[0]
    co2 = slf.shape[1]
    co = mu_b.shape[0]
    g0 = gacc[:NPAD]
    g1 = gacc[NPAD:]
    return pl.pallas_call(
        functools.partial(_gcnpost_body, co),
        grid=(n // blk,),
        in_specs=[pl.BlockSpec((blk, co2), lambda i: (i, 0)),
                  pl.BlockSpec((blk, co2), lambda i: (i, 0)),
                  pl.BlockSpec((blk, co2), lambda i: (i, 0)),
                  pl.BlockSpec((blk, 1), lambda i: (i, 0)),
                  pl.BlockSpec((1, co), lambda i: (0, 0)),
                  pl.BlockSpec((1, co), lambda i: (0, 0))],
        out_specs=[pl.BlockSpec((blk, co), lambda i: (i, 0)),
                   pl.BlockSpec((blk, co), lambda i: (i, 0))],
        out_shape=[jax.ShapeDtypeStruct((n, co), jnp.float32),
                   jax.ShapeDtypeStruct((n, co), jnp.float32)],
    )(g0, g1, slf, dv, mu_b.reshape(1, co), ls_b.reshape(1, co))


def _sc_mesh():
    return plsc.VectorSubcoreMesh(core_axis_name="c", subcore_axis_name="s")


_SC_PARAMS = pltpu.CompilerParams(use_tc_tiling_on_sc=False)


def _sc_gen(h, e, src3, dst3, mb, co):
    n, cop = h.shape
    nch = src3.shape[1]
    per_tile = nch * CHUNK
    co2 = 2 * co
    ng = co // 16

    @functools.partial(
        pl.kernel,
        out_type=jax.ShapeDtypeStruct((NCORES * NPAD, co2), jnp.float32),
        mesh=_sc_mesh(),
        compiler_params=_SC_PARAMS,
        scratch_types=[
            pltpu.VMEM((nch, CHUNK), jnp.int32),
            pltpu.VMEM((nch, CHUNK), jnp.int32),
            pltpu.VMEM((2, CHUNK, cop), jnp.float32),
            pltpu.VMEM((2, CHUNK, cop), jnp.float32),
            pltpu.VMEM((CHUNK, co2), jnp.float32),
            pltpu.VMEM((cop,), jnp.float32),
            pltpu.VMEM_SHARED((NPAD, co2), jnp.float32),
            pltpu.SemaphoreType.DMA,
            pltpu.SemaphoreType.DMA,
            pltpu.SemaphoreType.DMA,
            pltpu.SemaphoreType.DMA,
        ],
    )
    def kfn(h_hbm, e_hbm, src_hbm, dst_hbm, mb_hbm, out_hbm,
            src_v, dst_v, h_v, e_v, o_v, mb_v, acc, sh0, sh1, se0, se1):
        cid = lax.axis_index("c")
        sid = lax.axis_index("s")
        wid = sid * NCORES + cid
        zv = jnp.zeros((16,), jnp.float32)
        shs = (sh0, sh1)
        ses = (se0, se1)

        def zrow(i, carry):
            for j in range(co2 // 16):
                o_v[i, pl.ds(16 * j, 16)] = zv
            return carry

        lax.fori_loop(0, CHUNK, zrow, 0)
        for k in range(ZFULL):
            pltpu.sync_copy(o_v, acc.at[pl.ds(sid * ROWS_PER_SUB + k * CHUNK, CHUNK)])
        if ZREM:
            pltpu.sync_copy(o_v.at[pl.ds(0, ZREM)],
                            acc.at[pl.ds(sid * ROWS_PER_SUB + ZFULL * CHUNK, ZREM)])

        pltpu.sync_copy(mb_hbm, mb_v)
        pltpu.sync_copy(src_hbm.at[wid], src_v)
        pltpu.sync_copy(dst_hbm.at[wid], dst_v)
        mbs = [mb_v[pl.ds(16 * j, 16)] for j in range(ng)]
        plsc.subcore_barrier()

        def fetch(g, slot):
            base = wid * per_tile + g * CHUNK
            pltpu.async_copy(h_hbm.at[src_v.at[g]], h_v.at[slot], shs[slot])
            pltpu.async_copy(e_hbm.at[pl.ds(base, CHUNK)], e_v.at[slot], ses[slot])

        def wait(slot):
            pltpu.make_async_copy(h_hbm.at[src_v.at[0]], h_v.at[slot], shs[slot]).wait()
            pltpu.make_async_copy(e_hbm.at[pl.ds(0, CHUNK)], e_v.at[slot], ses[slot]).wait()

        def work(g, slot):
            wait(slot)

            def edge_body(i, c2):
                for j in range(ng):
                    m = h_v[slot, i, pl.ds(16 * j, 16)] + e_v[slot, i, pl.ds(16 * j, 16)]
                    m = jnp.maximum(m, 0.0) + EPS
                    t = jnp.exp(m - mbs[j])
                    o_v[i, pl.ds(16 * j, 16)] = t
                    o_v[i, pl.ds(co + 16 * j, 16)] = m * t
                return c2

            lax.fori_loop(0, CHUNK, edge_body, 0)
            pltpu.sync_copy(o_v, acc.at[dst_v.at[g]], add=True)

        fetch(0, 0)
        if nch > 1:
            fetch(1, 1)

        def pair_body(k, carry):
            ga = 2 * k
            work(ga, 0)

            @pl.when(ga + 2 < nch)
            def _():
                fetch(ga + 2, 0)

            work(ga + 1, 1)

            @pl.when(ga + 3 < nch)
            def _():
                fetch(ga + 3, 1)

            return carry

        lax.fori_loop(0, nch // 2, pair_body, 0)
        if nch % 2:
            work(nch - 1, (nch - 1) % 2)

        plsc.subcore_barrier()
        pltpu.sync_copy(acc.at[pl.ds(sid * ROWS_PER_SUB, ROWS_PER_SUB)],
                        out_hbm.at[pl.ds(cid * NPAD + sid * ROWS_PER_SUB, ROWS_PER_SUB)])

    return kfn(h, e, src3, dst3, mb)


def _sc_deg(dst3):
    nch = dst3.shape[1]

    @functools.partial(
        pl.kernel,
        out_type=jax.ShapeDtypeStruct((NCORES * NPAD, 16), jnp.float32),
        mesh=_sc_mesh(),
        compiler_params=_SC_PARAMS,
        scratch_types=[
            pltpu.VMEM((nch, CHUNK), jnp.int32),
            pltpu.VMEM((CHUNK, 16), jnp.float32),
            pltpu.VMEM_SHARED((NPAD, 16), jnp.float32),
        ],
    )
    def kfn(dst_hbm, out_hbm, dst_v, buf_v, acc):
        cid = lax.axis_index("c")
        sid = lax.axis_index("s")
        wid = sid * NCORES + cid
        zv = jnp.zeros((16,), jnp.float32)
        ov = jnp.ones((16,), jnp.float32)

        def zrow(i, carry):
            buf_v[i] = zv
            return carry

        lax.fori_loop(0, CHUNK, zrow, 0)
        for k in range(ZFULL):
            pltpu.sync_copy(buf_v, acc.at[pl.ds(sid * ROWS_PER_SUB + k * CHUNK, CHUNK)])
        if ZREM:
            pltpu.sync_copy(buf_v.at[pl.ds(0, ZREM)],
                            acc.at[pl.ds(sid * ROWS_PER_SUB + ZFULL * CHUNK, ZREM)])
        pltpu.sync_copy(dst_hbm.at[wid], dst_v)
        plsc.subcore_barrier()

        def orow(i, carry):
            buf_v[i] = ov
            return carry

        lax.fori_loop(0, CHUNK, orow, 0)

        def chunk_body(g, carry):
            pltpu.sync_copy(buf_v, acc.at[dst_v.at[g]], add=True)
            return carry

        lax.fori_loop(0, nch, chunk_body, 0)
        plsc.subcore_barrier()
        pltpu.sync_copy(acc.at[pl.ds(sid * ROWS_PER_SUB, ROWS_PER_SUB)],
                        out_hbm.at[pl.ds(cid * NPAD + sid * ROWS_PER_SUB, ROWS_PER_SUB)])

    return kfn(dst3)


def _sc_gcn(tab, src3, dst3):
    n, co = tab.shape
    nch = src3.shape[1]
    ng = co // 16

    @functools.partial(
        pl.kernel,
        out_type=jax.ShapeDtypeStruct((NCORES * NPAD, co), jnp.float32),
        mesh=_sc_mesh(),
        compiler_params=_SC_PARAMS,
        scratch_types=[
            pltpu.VMEM((nch, CHUNK), jnp.int32),
            pltpu.VMEM((nch, CHUNK), jnp.int32),
            pltpu.VMEM((2, CHUNK, co), jnp.float32),
            pltpu.VMEM_SHARED((NPAD, co), jnp.float32),
            pltpu.SemaphoreType.DMA,
            pltpu.SemaphoreType.DMA,
        ],
    )
    def kfn(tab_hbm, src_hbm, dst_hbm, out_hbm, src_v, dst_v, buf_v, acc,
            sg0, sg1):
        cid = lax.axis_index("c")
        sid = lax.axis_index("s")
        wid = sid * NCORES + cid
        zv = jnp.zeros((16,), jnp.float32)
        sgs = (sg0, sg1)

        def zrow(i, carry):
            for j in range(ng):
                buf_v[0, i, pl.ds(16 * j, 16)] = zv
            return carry

        lax.fori_loop(0, CHUNK, zrow, 0)
        for k in range(ZFULL):
            pltpu.sync_copy(buf_v.at[0],
                            acc.at[pl.ds(sid * ROWS_PER_SUB + k * CHUNK, CHUNK)])
        if ZREM:
            pltpu.sync_copy(buf_v.at[0].at[pl.ds(0, ZREM)],
                            acc.at[pl.ds(sid * ROWS_PER_SUB + ZFULL * CHUNK, ZREM)])
        pltpu.sync_copy(src_hbm.at[wid], src_v)
        pltpu.sync_copy(dst_hbm.at[wid], dst_v)
        plsc.subcore_barrier()

        def fetch(g, slot):
            pltpu.async_copy(tab_hbm.at[src_v.at[g]], buf_v.at[slot], sgs[slot])

        def work(g, slot):
            pltpu.make_async_copy(tab_hbm.at[src_v.at[0]], buf_v.at[slot],
                                  sgs[slot]).wait()
            pltpu.sync_copy(buf_v.at[slot], acc.at[dst_v.at[g]], add=True)

        fetch(0, 0)
        if nch > 1:
            fetch(1, 1)

        def pair_body(k, carry):
            ga = 2 * k
            work(ga, 0)

            @pl.when(ga + 2 < nch)
            def _():
                fetch(ga + 2, 0)

            work(ga + 1, 1)

            @pl.when(ga + 3 < nch)
            def _():
                fetch(ga + 3, 1)

            return carry

        lax.fori_loop(0, nch // 2, pair_body, 0)
        if nch % 2:
            work(nch - 1, (nch - 1) % 2)

        plsc.subcore_barrier()
        pltpu.sync_copy(acc.at[pl.ds(sid * ROWS_PER_SUB, ROWS_PER_SUB)],
                        out_hbm.at[pl.ds(cid * NPAD + sid * ROWS_PER_SUB, ROWS_PER_SUB)])

    return kfn(tab, src3, dst3)


def _gen_layer(x, ea, src, dst, Ws, bs, We, be, W1, b1, W2, b2):
    co = Ws.shape[1]
    splits = (co // 2, co - co // 2) if co > 70 else (co,)
    ns = len(splits)
    outs = _dense(x, Ws, bs, blk=2000, splits=splits)
    hs, hms = outs[:ns], outs[ns:]
    oute = _dense(ea, We, be, blk=10000, splits=splits)
    es, ems = oute[:ns], oute[ns:]
    pairs = []
    for h, e, hm, em in zip(hs, es, hms, ems):
        chn = h.shape[1]
        mb = (jnp.maximum(hm + em, 0.0) + EPS).reshape(chn)
        acc = _sc_gen(h, e, src, dst, mb, chn)
        pairs.append((acc, h))
    return _mlp(pairs, W1, b1, W2, b2)


def kernel(x, edge_index, edge_attr, g1_Ws, g1_bs, g1_We, g1_be, g1_W1,
           g1_b1, g1_W2, g1_b2, g2_Ws, g2_bs, g2_We, g2_be, g2_W1, g2_b1,
           g2_W2, g2_b2, mu_W, mu_b, ls_W, ls_b):
    ee = edge_index.shape[1]
    nch = ee // (NW * CHUNK)
    src = edge_index[0].reshape(NW, nch, CHUNK)
    dst = edge_index[1].reshape(NW, nch, CHUNK)

    dega = _sc_deg(dst)

    x2 = _gen_layer(x, edge_attr, src, dst, g1_Ws, g1_bs, g1_We, g1_be,
                    g1_W1, g1_b1, g1_W2, g1_b2)
    x3 = _gen_layer(x2, edge_attr, src, dst, g2_Ws, g2_bs, g2_We, g2_be,
                    g2_W1, g2_b1, g2_W2, g2_b2)

    wcat = jnp.concatenate([mu_W, ls_W], axis=1)
    tab, slf, dv = _gcn_pre(x3, wcat, dega)
    gacc = _sc_gcn(tab, src, dst)
    return _gcn_post(gacc, slf, dv, mu_b, ls_b)


# speedup vs baseline: 13.5299x; 1.8211x over previous
import functools
import math

import jax
import jax.numpy as jnp
from jax import lax
from jax.experimental import pallas as pl
from jax.experimental.pallas import tpu as pltpu
from jax.experimental.pallas import tpu_sc as plsc

EPS = 1e-7
NPAD = 10016
CHUNK = 80
NCORES = 2
NSUB = 16
NW = NCORES * NSUB
ROWS_PER_SUB = NPAD // NSUB
ZFULL = ROWS_PER_SUB // CHUNK
ZREM = ROWS_PER_SUB % CHUNK


def _dense_body(bounds, x_ref, w_ref, b_ref, *refs):
    ns = len(bounds) - 1
    i = pl.program_id(0)
    o = jnp.dot(x_ref[...], w_ref[...], preferred_element_type=jnp.float32)
    o = o + b_ref[...]
    for s in range(ns):
        lo, hi = bounds[s], bounds[s + 1]
        blkc = o[:, lo:hi]
        refs[s][...] = blkc
        bm = jnp.max(blkc, axis=0, keepdims=True)
        m_ref = refs[ns + s]

        @pl.when(i == 0)
        def _():
            m_ref[...] = bm

        @pl.when(i != 0)
        def _():
            m_ref[...] = jnp.maximum(m_ref[...], bm)


def _dense(x, w, b, blk, splits=None):
    n, k = x.shape
    co = w.shape[1]
    if splits is None:
        splits = (co,)
    bounds = [0]
    for s in splits:
        bounds.append(bounds[-1] + s)
    out_shape = ([jax.ShapeDtypeStruct((n, s), jnp.float32) for s in splits]
                 + [jax.ShapeDtypeStruct((1, s), jnp.float32) for s in splits])
    out_specs = ([pl.BlockSpec((blk, s), lambda i: (i, 0)) for s in splits]
                 + [pl.BlockSpec((1, s), lambda i: (0, 0)) for s in splits])
    return pl.pallas_call(
        functools.partial(_dense_body, tuple(bounds)),
        grid=(n // blk,),
        in_specs=[pl.BlockSpec((blk, k), lambda i: (i, 0)),
                  pl.BlockSpec((k, co), lambda i: (0, 0)),
                  pl.BlockSpec((1, co), lambda i: (0, 0))],
        out_specs=out_specs,
        out_shape=out_shape,
    )(x, w, b.reshape(1, co))


def _mlp_body(nh, w1_ref, b1_ref, w2_ref, b2_ref, *refs, o_ref):
    parts = []
    for s in range(nh):
        a0_ref, a1_ref, h_ref = refs[3 * s], refs[3 * s + 1], refs[3 * s + 2]
        ch = h_ref.shape[1]
        sm = a0_ref[:, :ch] + a1_ref[:, :ch]
        wm = a0_ref[:, ch:] + a1_ref[:, ch:]
        parts.append(wm / (sm + 1e-16) + h_ref[...])
    out = parts[0] if nh == 1 else jnp.concatenate(parts, axis=1)
    t = jnp.dot(out, w1_ref[...], preferred_element_type=jnp.float32) + b1_ref[...]
    t = jnp.maximum(t, 0.0)
    t = jnp.dot(t, w2_ref[...], preferred_element_type=jnp.float32) + b2_ref[...]
    o_ref[...] = jnp.maximum(t, 0.0)


def _mlp_wrap(nh, *args):
    o_ref = args[-1]
    w1_ref, b1_ref, w2_ref, b2_ref = args[:4]
    _mlp_body(nh, w1_ref, b1_ref, w2_ref, b2_ref, *args[4:-1], o_ref=o_ref)


def _mlp(pairs, w1, b1, w2, b2, blk=2000):
    co = w1.shape[0]
    ch_mlp = w1.shape[1]
    n = pairs[0][1].shape[0]
    ins = [w1, b1.reshape(1, ch_mlp), w2, b2.reshape(1, co)]
    in_specs = [pl.BlockSpec((co, ch_mlp), lambda i: (0, 0)),
                pl.BlockSpec((1, ch_mlp), lambda i: (0, 0)),
                pl.BlockSpec((ch_mlp, co), lambda i: (0, 0)),
                pl.BlockSpec((1, co), lambda i: (0, 0))]
    for acc, h in pairs:
        ch2 = acc.shape[1]
        ch = h.shape[1]
        ins += [acc[:NPAD], acc[NPAD:], h]
        in_specs += [pl.BlockSpec((blk, ch2), lambda i: (i, 0)),
                     pl.BlockSpec((blk, ch2), lambda i: (i, 0)),
                     pl.BlockSpec((blk, ch), lambda i: (i, 0))]
    return pl.pallas_call(
        functools.partial(_mlp_wrap, len(pairs)),
        grid=(n // blk,),
        in_specs=in_specs,
        out_specs=pl.BlockSpec((blk, co), lambda i: (i, 0)),
        out_shape=jax.ShapeDtypeStruct((n, co), jnp.float32),
    )(*ins)


def _gcnpre_body(x_ref, w_ref, d0_ref, d1_ref, tab_ref, slf_ref, dv_ref):
    deg = 1.0 + d0_ref[:, 0:1] + d1_ref[:, 0:1]
    dinv = lax.rsqrt(deg)
    hc = jnp.dot(x_ref[...], w_ref[...], preferred_element_type=jnp.float32)
    tab_ref[...] = hc * dinv
    slf_ref[...] = hc * (dinv * dinv)
    dv_ref[...] = dinv


def _gcn_pre(x, wcat, dega, blk=2000):
    n, k = x.shape
    co2 = wcat.shape[1]
    d0 = dega[:NPAD]
    d1 = dega[NPAD:]
    return pl.pallas_call(
        _gcnpre_body,
        grid=(n // blk,),
        in_specs=[pl.BlockSpec((blk, k), lambda i: (i, 0)),
                  pl.BlockSpec((k, co2), lambda i: (0, 0)),
                  pl.BlockSpec((blk, 16), lambda i: (i, 0)),
                  pl.BlockSpec((blk, 16), lambda i: (i, 0))],
        out_specs=[pl.BlockSpec((blk, co2), lambda i: (i, 0)),
                   pl.BlockSpec((blk, co2), lambda i: (i, 0)),
                   pl.BlockSpec((blk, 1), lambda i: (i, 0))],
        out_shape=[jax.ShapeDtypeStruct((n, co2), jnp.float32),
                   jax.ShapeDtypeStruct((n, co2), jnp.float32),
                   jax.ShapeDtypeStruct((n, 1), jnp.float32)],
    )(x, wcat, d0, d1)


def _gcnpost_body(co, g0_ref, g1_ref, slf_ref, dv_ref, bm_ref, bl_ref, mu_ref, ls_ref):
    agg = g0_ref[...] + g1_ref[...]
    out = dv_ref[...] * agg + slf_ref[...]
    mu_ref[...] = out[:, :co] + bm_ref[...]
    ls_ref[...] = out[:, co:2 * co] + bl_ref[...]


def _gcn_post(gacc, slf, dv, mu_b, ls_b, blk=2000):
    n = slf.shape[0]
    co2 = slf.shape[1]
    co = mu_b.shape[0]
    g0 = gacc[:NPAD]
    g1 = gacc[NPAD:]
    return pl.pallas_call(
        functools.partial(_gcnpost_body, co),
        grid=(n // blk,),
        in_specs=[pl.BlockSpec((blk, co2), lambda i: (i, 0)),
                  pl.BlockSpec((blk, co2), lambda i: (i, 0)),
                  pl.BlockSpec((blk, co2), lambda i: (i, 0)),
                  pl.BlockSpec((blk, 1), lambda i: (i, 0)),
                  pl.BlockSpec((1, co), lambda i: (0, 0)),
                  pl.BlockSpec((1, co), lambda i: (0, 0))],
        out_specs=[pl.BlockSpec((blk, co), lambda i: (i, 0)),
                   pl.BlockSpec((blk, co), lambda i: (i, 0))],
        out_shape=[jax.ShapeDtypeStruct((n, co), jnp.float32),
                   jax.ShapeDtypeStruct((n, co), jnp.float32)],
    )(g0, g1, slf, dv, mu_b.reshape(1, co), ls_b.reshape(1, co))


def _sc_mesh():
    return plsc.VectorSubcoreMesh(core_axis_name="c", subcore_axis_name="s")


_SC_PARAMS = pltpu.CompilerParams(use_tc_tiling_on_sc=False)


def _sc_gen(h, e, src3, dst3, mb, co):
    n, cop = h.shape
    nch = src3.shape[1]
    per_tile = nch * CHUNK
    co2 = 2 * co
    ng = co // 16

    @functools.partial(
        pl.kernel,
        out_type=jax.ShapeDtypeStruct((NCORES * NPAD, co2), jnp.float32),
        mesh=_sc_mesh(),
        compiler_params=_SC_PARAMS,
        scratch_types=[
            pltpu.VMEM((nch, CHUNK), jnp.int32),
            pltpu.VMEM((nch, CHUNK), jnp.int32),
            pltpu.VMEM((2, CHUNK, cop), jnp.float32),
            pltpu.VMEM((2, CHUNK, cop), jnp.float32),
            pltpu.VMEM((2, CHUNK, co2), jnp.float32),
            pltpu.VMEM((cop,), jnp.float32),
            pltpu.VMEM_SHARED((NPAD, co2), jnp.float32),
            pltpu.SemaphoreType.DMA,
            pltpu.SemaphoreType.DMA,
            pltpu.SemaphoreType.DMA,
            pltpu.SemaphoreType.DMA,
            pltpu.SemaphoreType.DMA,
            pltpu.SemaphoreType.DMA,
        ],
    )
    def kfn(h_hbm, e_hbm, src_hbm, dst_hbm, mb_hbm, out_hbm,
            src_v, dst_v, h_v, e_v, o_v, mb_v, acc,
            sh0, sh1, se0, se1, ss0, ss1):
        cid = lax.axis_index("c")
        sid = lax.axis_index("s")
        wid = sid * NCORES + cid
        zv = jnp.zeros((16,), jnp.float32)
        shs = (sh0, sh1)
        ses = (se0, se1)
        sss = (ss0, ss1)

        def zrow(i, carry):
            for j in range(co2 // 16):
                o_v[0, i, pl.ds(16 * j, 16)] = zv
            return carry

        lax.fori_loop(0, CHUNK, zrow, 0)
        for k in range(ZFULL):
            pltpu.sync_copy(o_v.at[0],
                            acc.at[pl.ds(sid * ROWS_PER_SUB + k * CHUNK, CHUNK)])
        if ZREM:
            pltpu.sync_copy(o_v.at[0].at[pl.ds(0, ZREM)],
                            acc.at[pl.ds(sid * ROWS_PER_SUB + ZFULL * CHUNK, ZREM)])

        pltpu.sync_copy(mb_hbm, mb_v)
        pltpu.sync_copy(src_hbm.at[wid], src_v)
        pltpu.sync_copy(dst_hbm.at[wid], dst_v)
        mbs = [mb_v[pl.ds(16 * j, 16)] for j in range(ng)]
        plsc.subcore_barrier()

        def fetch(g, slot):
            base = wid * per_tile + g * CHUNK
            pltpu.async_copy(h_hbm.at[src_v.at[g]], h_v.at[slot], shs[slot])
            pltpu.async_copy(e_hbm.at[pl.ds(base, CHUNK)], e_v.at[slot], ses[slot])

        def wait(slot):
            pltpu.make_async_copy(h_hbm.at[src_v.at[0]], h_v.at[slot], shs[slot]).wait()
            pltpu.make_async_copy(e_hbm.at[pl.ds(0, CHUNK)], e_v.at[slot], ses[slot]).wait()

        def wait_scat(slot):
            pltpu.make_async_copy(o_v.at[slot], acc.at[dst_v.at[0]], sss[slot]).wait()

        def work(g, slot, first):
            wait(slot)
            if not first:
                wait_scat(slot)

            @plsc.parallel_loop(0, CHUNK, unroll=8)
            def edge_body(i):
                for j in range(ng):
                    m = h_v[slot, i, pl.ds(16 * j, 16)] + e_v[slot, i, pl.ds(16 * j, 16)]
                    m = jnp.maximum(m, 0.0) + EPS
                    t = jnp.exp(m - mbs[j])
                    o_v[slot, i, pl.ds(16 * j, 16)] = t
                    o_v[slot, i, pl.ds(co + 16 * j, 16)] = m * t

            pltpu.async_copy(o_v.at[slot], acc.at[dst_v.at[g]], sss[slot],
                             add=True)

        fetch(0, 0)
        if nch > 1:
            fetch(1, 1)
        work(0, 0, True)
        if nch > 2:
            fetch(2, 0)
        work(1, 1, True)
        if nch > 3:
            fetch(3, 1)

        def pair_body(k, carry):
            ga = 2 * k + 2
            work(ga, 0, False)

            @pl.when(ga + 2 < nch)
            def _():
                fetch(ga + 2, 0)

            work(ga + 1, 1, False)

            @pl.when(ga + 3 < nch)
            def _():
                fetch(ga + 3, 1)

            return carry

        lax.fori_loop(0, (nch - 2) // 2, pair_body, 0)
        if nch % 2:
            work(nch - 1, (nch - 1) % 2, False)
        wait_scat(0)
        wait_scat(1)

        plsc.subcore_barrier()
        pltpu.sync_copy(acc.at[pl.ds(sid * ROWS_PER_SUB, ROWS_PER_SUB)],
                        out_hbm.at[pl.ds(cid * NPAD + sid * ROWS_PER_SUB, ROWS_PER_SUB)])

    return kfn(h, e, src3, dst3, mb)


def _sc_deg(dst3):
    nch = dst3.shape[1]

    @functools.partial(
        pl.kernel,
        out_type=jax.ShapeDtypeStruct((NCORES * NPAD, 16), jnp.float32),
        mesh=_sc_mesh(),
        compiler_params=_SC_PARAMS,
        scratch_types=[
            pltpu.VMEM((nch, CHUNK), jnp.int32),
            pltpu.VMEM((CHUNK, 16), jnp.float32),
            pltpu.VMEM_SHARED((NPAD, 16), jnp.float32),
        ],
    )
    def kfn(dst_hbm, out_hbm, dst_v, buf_v, acc):
        cid = lax.axis_index("c")
        sid = lax.axis_index("s")
        wid = sid * NCORES + cid
        zv = jnp.zeros((16,), jnp.float32)
        ov = jnp.ones((16,), jnp.float32)

        def zrow(i, carry):
            buf_v[i] = zv
            return carry

        lax.fori_loop(0, CHUNK, zrow, 0)
        for k in range(ZFULL):
            pltpu.sync_copy(buf_v, acc.at[pl.ds(sid * ROWS_PER_SUB + k * CHUNK, CHUNK)])
        if ZREM:
            pltpu.sync_copy(buf_v.at[pl.ds(0, ZREM)],
                            acc.at[pl.ds(sid * ROWS_PER_SUB + ZFULL * CHUNK, ZREM)])
        pltpu.sync_copy(dst_hbm.at[wid], dst_v)
        plsc.subcore_barrier()

        def orow(i, carry):
            buf_v[i] = ov
            return carry

        lax.fori_loop(0, CHUNK, orow, 0)

        def chunk_body(g, carry):
            pltpu.sync_copy(buf_v, acc.at[dst_v.at[g]], add=True)
            return carry

        lax.fori_loop(0, nch, chunk_body, 0)
        plsc.subcore_barrier()
        pltpu.sync_copy(acc.at[pl.ds(sid * ROWS_PER_SUB, ROWS_PER_SUB)],
                        out_hbm.at[pl.ds(cid * NPAD + sid * ROWS_PER_SUB, ROWS_PER_SUB)])

    return kfn(dst3)


def _sc_gcn(tab, src3, dst3):
    n, co = tab.shape
    nch = src3.shape[1]
    ng = co // 16

    @functools.partial(
        pl.kernel,
        out_type=jax.ShapeDtypeStruct((NCORES * NPAD, co), jnp.float32),
        mesh=_sc_mesh(),
        compiler_params=_SC_PARAMS,
        scratch_types=[
            pltpu.VMEM((nch, CHUNK), jnp.int32),
            pltpu.VMEM((nch, CHUNK), jnp.int32),
            pltpu.VMEM((2, CHUNK, co), jnp.float32),
            pltpu.VMEM_SHARED((NPAD, co), jnp.float32),
            pltpu.SemaphoreType.DMA,
            pltpu.SemaphoreType.DMA,
        ],
    )
    def kfn(tab_hbm, src_hbm, dst_hbm, out_hbm, src_v, dst_v, buf_v, acc,
            sg0, sg1):
        cid = lax.axis_index("c")
        sid = lax.axis_index("s")
        wid = sid * NCORES + cid
        zv = jnp.zeros((16,), jnp.float32)
        sgs = (sg0, sg1)

        def zrow(i, carry):
            for j in range(ng):
                buf_v[0, i, pl.ds(16 * j, 16)] = zv
            return carry

        lax.fori_loop(0, CHUNK, zrow, 0)
        for k in range(ZFULL):
            pltpu.sync_copy(buf_v.at[0],
                            acc.at[pl.ds(sid * ROWS_PER_SUB + k * CHUNK, CHUNK)])
        if ZREM:
            pltpu.sync_copy(buf_v.at[0].at[pl.ds(0, ZREM)],
                            acc.at[pl.ds(sid * ROWS_PER_SUB + ZFULL * CHUNK, ZREM)])
        pltpu.sync_copy(src_hbm.at[wid], src_v)
        pltpu.sync_copy(dst_hbm.at[wid], dst_v)
        plsc.subcore_barrier()

        def fetch(g, slot):
            pltpu.async_copy(tab_hbm.at[src_v.at[g]], buf_v.at[slot], sgs[slot])

        def work(g, slot):
            pltpu.make_async_copy(tab_hbm.at[src_v.at[0]], buf_v.at[slot],
                                  sgs[slot]).wait()
            pltpu.sync_copy(buf_v.at[slot], acc.at[dst_v.at[g]], add=True)

        fetch(0, 0)
        if nch > 1:
            fetch(1, 1)

        def pair_body(k, carry):
            ga = 2 * k
            work(ga, 0)

            @pl.when(ga + 2 < nch)
            def _():
                fetch(ga + 2, 0)

            work(ga + 1, 1)

            @pl.when(ga + 3 < nch)
            def _():
                fetch(ga + 3, 1)

            return carry

        lax.fori_loop(0, nch // 2, pair_body, 0)
        if nch % 2:
            work(nch - 1, (nch - 1) % 2)

        plsc.subcore_barrier()
        pltpu.sync_copy(acc.at[pl.ds(sid * ROWS_PER_SUB, ROWS_PER_SUB)],
                        out_hbm.at[pl.ds(cid * NPAD + sid * ROWS_PER_SUB, ROWS_PER_SUB)])

    return kfn(tab, src3, dst3)


def _gen_layer(x, ea, src, dst, Ws, bs, We, be, W1, b1, W2, b2):
    co = Ws.shape[1]
    splits = (co // 2, co - co // 2) if co > 48 else (co,)
    ns = len(splits)
    outs = _dense(x, Ws, bs, blk=2000, splits=splits)
    hs, hms = outs[:ns], outs[ns:]
    oute = _dense(ea, We, be, blk=10000, splits=splits)
    es, ems = oute[:ns], oute[ns:]
    pairs = []
    for h, e, hm, em in zip(hs, es, hms, ems):
        chn = h.shape[1]
        mb = (jnp.maximum(hm + em, 0.0) + EPS).reshape(chn)
        acc = _sc_gen(h, e, src, dst, mb, chn)
        pairs.append((acc, h))
    return _mlp(pairs, W1, b1, W2, b2)


def kernel(x, edge_index, edge_attr, g1_Ws, g1_bs, g1_We, g1_be, g1_W1,
           g1_b1, g1_W2, g1_b2, g2_Ws, g2_bs, g2_We, g2_be, g2_W1, g2_b1,
           g2_W2, g2_b2, mu_W, mu_b, ls_W, ls_b):
    ee = edge_index.shape[1]
    nch = ee // (NW * CHUNK)
    src = edge_index[0].reshape(NW, nch, CHUNK)
    dst = edge_index[1].reshape(NW, nch, CHUNK)

    dega = _sc_deg(dst)

    x2 = _gen_layer(x, edge_attr, src, dst, g1_Ws, g1_bs, g1_We, g1_be,
                    g1_W1, g1_b1, g1_W2, g1_b2)
    x3 = _gen_layer(x2, edge_attr, src, dst, g2_Ws, g2_bs, g2_We, g2_be,
                    g2_W1, g2_b1, g2_W2, g2_b2)

    wcat = jnp.concatenate([mu_W, ls_W], axis=1)
    tab, slf, dv = _gcn_pre(x3, wcat, dega)
    gacc = _sc_gcn(tab, src, dst)
    return _gcn_post(gacc, slf, dv, mu_b, ls_b)


# speedup vs baseline: 20.4829x; 1.5139x over previous
import functools
import math

import jax
import jax.numpy as jnp
from jax import lax
from jax.experimental import pallas as pl
from jax.experimental.pallas import tpu as pltpu
from jax.experimental.pallas import tpu_sc as plsc

EPS = 1e-7
NPAD = 10016
CHUNK = 80
NCORES = 2
NSUB = 16
NW = NCORES * NSUB
ROWS_PER_SUB = NPAD // NSUB
ZFULL = ROWS_PER_SUB // CHUNK
ZREM = ROWS_PER_SUB % CHUNK


def _dense_body(bounds, flat, x_ref, w_ref, b_ref, *refs):
    ns = len(bounds) - 1
    i = pl.program_id(0)
    o = jnp.dot(x_ref[...], w_ref[...], preferred_element_type=jnp.float32)
    o = o + b_ref[...]
    for s in range(ns):
        lo, hi = bounds[s], bounds[s + 1]
        blkc = o[:, lo:hi]
        if flat:
            refs[s][...] = blkc.reshape(-1)
        else:
            refs[s][...] = blkc
        bm = jnp.max(blkc, axis=0, keepdims=True)
        m_ref = refs[ns + s]

        @pl.when(i == 0)
        def _():
            m_ref[...] = bm

        @pl.when(i != 0)
        def _():
            m_ref[...] = jnp.maximum(m_ref[...], bm)


def _dense(x, w, b, blk, splits=None, flat=False):
    n, k = x.shape
    co = w.shape[1]
    if splits is None:
        splits = (co,)
    bounds = [0]
    for s in splits:
        bounds.append(bounds[-1] + s)
    if flat:
        out_shape = [jax.ShapeDtypeStruct((n * s,), jnp.float32) for s in splits]
        out_specs = [pl.BlockSpec((blk * s, ), lambda i: (i,)) for s in splits]
    else:
        out_shape = [jax.ShapeDtypeStruct((n, s), jnp.float32) for s in splits]
        out_specs = [pl.BlockSpec((blk, s), lambda i: (i, 0)) for s in splits]
    out_shape += [jax.ShapeDtypeStruct((1, s), jnp.float32) for s in splits]
    out_specs += [pl.BlockSpec((1, s), lambda i: (0, 0)) for s in splits]
    return pl.pallas_call(
        functools.partial(_dense_body, tuple(bounds), flat),
        grid=(n // blk,),
        in_specs=[pl.BlockSpec((blk, k), lambda i: (i, 0)),
                  pl.BlockSpec((k, co), lambda i: (0, 0)),
                  pl.BlockSpec((1, co), lambda i: (0, 0))],
        out_specs=out_specs,
        out_shape=out_shape,
    )(x, w, b.reshape(1, co))


def _mlp_body(nh, w1_ref, b1_ref, w2_ref, b2_ref, *refs, o_ref):
    parts = []
    for s in range(nh):
        a0_ref, a1_ref, h_ref = refs[3 * s], refs[3 * s + 1], refs[3 * s + 2]
        ch = h_ref.shape[1]
        sm = a0_ref[:, :ch] + a1_ref[:, :ch]
        wm = a0_ref[:, ch:] + a1_ref[:, ch:]
        parts.append(wm / (sm + 1e-16) + h_ref[...])
    out = parts[0] if nh == 1 else jnp.concatenate(parts, axis=1)
    t = jnp.dot(out, w1_ref[...], preferred_element_type=jnp.float32) + b1_ref[...]
    t = jnp.maximum(t, 0.0)
    t = jnp.dot(t, w2_ref[...], preferred_element_type=jnp.float32) + b2_ref[...]
    o_ref[...] = jnp.maximum(t, 0.0)


def _mlp_wrap(nh, *args):
    o_ref = args[-1]
    w1_ref, b1_ref, w2_ref, b2_ref = args[:4]
    _mlp_body(nh, w1_ref, b1_ref, w2_ref, b2_ref, *args[4:-1], o_ref=o_ref)


def _mlp(pairs, w1, b1, w2, b2, blk=2000):
    co = w1.shape[0]
    ch_mlp = w1.shape[1]
    n = pairs[0][1].shape[0]
    ins = [w1, b1.reshape(1, ch_mlp), w2, b2.reshape(1, co)]
    in_specs = [pl.BlockSpec((co, ch_mlp), lambda i: (0, 0)),
                pl.BlockSpec((1, ch_mlp), lambda i: (0, 0)),
                pl.BlockSpec((ch_mlp, co), lambda i: (0, 0)),
                pl.BlockSpec((1, co), lambda i: (0, 0))]
    for acc, h in pairs:
        ch2 = acc.shape[1]
        ch = h.shape[1]
        ins += [acc[:NPAD], acc[NPAD:], h]
        in_specs += [pl.BlockSpec((blk, ch2), lambda i: (i, 0)),
                     pl.BlockSpec((blk, ch2), lambda i: (i, 0)),
                     pl.BlockSpec((blk, ch), lambda i: (i, 0))]
    return pl.pallas_call(
        functools.partial(_mlp_wrap, len(pairs)),
        grid=(n // blk,),
        in_specs=in_specs,
        out_specs=pl.BlockSpec((blk, co), lambda i: (i, 0)),
        out_shape=jax.ShapeDtypeStruct((n, co), jnp.float32),
    )(*ins)


def _gcnpre_body(x_ref, w_ref, d0_ref, d1_ref, tab_ref, slf_ref, dv_ref):
    deg = 1.0 + d0_ref[:, 0:1] + d1_ref[:, 0:1]
    dinv = lax.rsqrt(deg)
    hc = jnp.dot(x_ref[...], w_ref[...], preferred_element_type=jnp.float32)
    tab_ref[...] = hc * dinv
    slf_ref[...] = hc * (dinv * dinv)
    dv_ref[...] = dinv


def _gcn_pre(x, wcat, dega, blk=2000):
    n, k = x.shape
    co2 = wcat.shape[1]
    d0 = dega[:NPAD]
    d1 = dega[NPAD:]
    return pl.pallas_call(
        _gcnpre_body,
        grid=(n // blk,),
        in_specs=[pl.BlockSpec((blk, k), lambda i: (i, 0)),
                  pl.BlockSpec((k, co2), lambda i: (0, 0)),
                  pl.BlockSpec((blk, 16), lambda i: (i, 0)),
                  pl.BlockSpec((blk, 16), lambda i: (i, 0))],
        out_specs=[pl.BlockSpec((blk, co2), lambda i: (i, 0)),
                   pl.BlockSpec((blk, co2), lambda i: (i, 0)),
                   pl.BlockSpec((blk, 1), lambda i: (i, 0))],
        out_shape=[jax.ShapeDtypeStruct((n, co2), jnp.float32),
                   jax.ShapeDtypeStruct((n, co2), jnp.float32),
                   jax.ShapeDtypeStruct((n, 1), jnp.float32)],
    )(x, wcat, d0, d1)


def _gcnpost_body(co, g0_ref, g1_ref, slf_ref, dv_ref, bm_ref, bl_ref, mu_ref, ls_ref):
    agg = g0_ref[...] + g1_ref[...]
    out = dv_ref[...] * agg + slf_ref[...]
    mu_ref[...] = out[:, :co] + bm_ref[...]
    ls_ref[...] = out[:, co:2 * co] + bl_ref[...]


def _gcn_post(gacc, slf, dv, mu_b, ls_b, blk=2000):
    n = slf.shape[0]
    co2 = slf.shape[1]
    co = mu_b.shape[0]
    g0 = gacc[:NPAD]
    g1 = gacc[NPAD:]
    return pl.pallas_call(
        functools.partial(_gcnpost_body, co),
        grid=(n // blk,),
        in_specs=[pl.BlockSpec((blk, co2), lambda i: (i, 0)),
                  pl.BlockSpec((blk, co2), lambda i: (i, 0)),
                  pl.BlockSpec((blk, co2), lambda i: (i, 0)),
                  pl.BlockSpec((blk, 1), lambda i: (i, 0)),
                  pl.BlockSpec((1, co), lambda i: (0, 0)),
                  pl.BlockSpec((1, co), lambda i: (0, 0))],
        out_specs=[pl.BlockSpec((blk, co), lambda i: (i, 0)),
                   pl.BlockSpec((blk, co), lambda i: (i, 0))],
        out_shape=[jax.ShapeDtypeStruct((n, co), jnp.float32),
                   jax.ShapeDtypeStruct((n, co), jnp.float32)],
    )(g0, g1, slf, dv, mu_b.reshape(1, co), ls_b.reshape(1, co))


def _sc_mesh():
    return plsc.VectorSubcoreMesh(core_axis_name="c", subcore_axis_name="s")


_SC_PARAMS = pltpu.CompilerParams(use_tc_tiling_on_sc=False)


def _sc_gen(h, e, src3, dst3, mb, co, col0):
    n, cop = h.shape
    nch = src3.shape[1]
    per_tile = nch * CHUNK
    co2 = 2 * co
    ng = co // 16

    @functools.partial(
        pl.kernel,
        out_type=jax.ShapeDtypeStruct((NCORES * NPAD, co2), jnp.float32),
        mesh=_sc_mesh(),
        compiler_params=_SC_PARAMS,
        scratch_types=[
            pltpu.VMEM((nch, CHUNK), jnp.int32),
            pltpu.VMEM((nch, CHUNK), jnp.int32),
            pltpu.VMEM((2, CHUNK, cop), jnp.float32),
            pltpu.VMEM((2, CHUNK, cop), jnp.float32),
            pltpu.VMEM((2, CHUNK, co2), jnp.float32),
            pltpu.VMEM((cop,), jnp.float32),
            pltpu.VMEM_SHARED((NPAD, co2), jnp.float32),
            pltpu.SemaphoreType.DMA,
            pltpu.SemaphoreType.DMA,
            pltpu.SemaphoreType.DMA,
            pltpu.SemaphoreType.DMA,
            pltpu.SemaphoreType.DMA,
            pltpu.SemaphoreType.DMA,
        ],
    )
    def kfn(h_hbm, e_hbm, src_hbm, dst_hbm, mb_hbm, out_hbm,
            src_v, dst_v, h_v, e_v, o_v, mb_v, acc,
            sh0, sh1, se0, se1, ss0, ss1):
        cid = lax.axis_index("c")
        sid = lax.axis_index("s")
        wid = sid * NCORES + cid
        zv = jnp.zeros((16,), jnp.float32)
        shs = (sh0, sh1)
        ses = (se0, se1)
        sss = (ss0, ss1)

        def zrow(i, carry):
            for j in range(co2 // 16):
                o_v[0, i, pl.ds(16 * j, 16)] = zv
            return carry

        lax.fori_loop(0, CHUNK, zrow, 0)
        for k in range(ZFULL):
            pltpu.sync_copy(o_v.at[0],
                            acc.at[pl.ds(sid * ROWS_PER_SUB + k * CHUNK, CHUNK)])
        if ZREM:
            pltpu.sync_copy(o_v.at[0].at[pl.ds(0, ZREM)],
                            acc.at[pl.ds(sid * ROWS_PER_SUB + ZFULL * CHUNK, ZREM)])

        pltpu.sync_copy(mb_hbm, mb_v)
        pltpu.sync_copy(src_hbm.at[wid], src_v)
        pltpu.sync_copy(dst_hbm.at[wid], dst_v)
        mbs = [mb_v[pl.ds(16 * j, 16)] for j in range(ng)]
        plsc.subcore_barrier()

        def fetch(g, slot):
            base = wid * per_tile + g * CHUNK
            pltpu.async_copy(h_hbm.at[src_v.at[g]], h_v.at[slot], shs[slot])
            pltpu.async_copy(e_hbm.at[pl.ds(base, CHUNK), pl.ds(col0, cop)],
                             e_v.at[slot], ses[slot])

        def wait(slot):
            pltpu.make_async_copy(h_hbm.at[src_v.at[0]], h_v.at[slot], shs[slot]).wait()
            pltpu.make_async_copy(e_hbm.at[pl.ds(0, CHUNK), pl.ds(col0, cop)],
                                  e_v.at[slot], ses[slot]).wait()

        def wait_scat(slot):
            pltpu.make_async_copy(o_v.at[slot], acc.at[dst_v.at[0]], sss[slot]).wait()

        def work(g, slot, first):
            wait(slot)
            if not first:
                wait_scat(slot)

            @plsc.parallel_loop(0, CHUNK, unroll=8)
            def edge_body(i):
                for j in range(ng):
                    m = (h_v[slot, i, pl.ds(16 * j, 16)]
                         + e_v[slot, i, pl.ds(16 * j, 16)])
                    m = jnp.maximum(m, 0.0) + EPS
                    t = jnp.exp(m - mbs[j])
                    o_v[slot, i, pl.ds(16 * j, 16)] = t
                    o_v[slot, i, pl.ds(co + 16 * j, 16)] = m * t

            pltpu.async_copy(o_v.at[slot], acc.at[dst_v.at[g]], sss[slot],
                             add=True)

        fetch(0, 0)
        if nch > 1:
            fetch(1, 1)
        work(0, 0, True)
        if nch > 2:
            fetch(2, 0)
        work(1, 1, True)
        if nch > 3:
            fetch(3, 1)

        def pair_body(k, carry):
            ga = 2 * k + 2
            work(ga, 0, False)

            @pl.when(ga + 2 < nch)
            def _():
                fetch(ga + 2, 0)

            work(ga + 1, 1, False)

            @pl.when(ga + 3 < nch)
            def _():
                fetch(ga + 3, 1)

            return carry

        lax.fori_loop(0, (nch - 2) // 2, pair_body, 0)
        if nch % 2:
            work(nch - 1, (nch - 1) % 2, False)
        wait_scat(0)
        wait_scat(1)

        plsc.subcore_barrier()
        pltpu.sync_copy(acc.at[pl.ds(sid * ROWS_PER_SUB, ROWS_PER_SUB)],
                        out_hbm.at[pl.ds(cid * NPAD + sid * ROWS_PER_SUB, ROWS_PER_SUB)])

    return kfn(h, e, src3, dst3, mb)


def _sc_deg(dst3):
    nch = dst3.shape[1]

    @functools.partial(
        pl.kernel,
        out_type=jax.ShapeDtypeStruct((NCORES * NPAD, 16), jnp.float32),
        mesh=_sc_mesh(),
        compiler_params=_SC_PARAMS,
        scratch_types=[
            pltpu.VMEM((nch, CHUNK), jnp.int32),
            pltpu.VMEM((CHUNK, 16), jnp.float32),
            pltpu.VMEM_SHARED((NPAD, 16), jnp.float32),
        ],
    )
    def kfn(dst_hbm, out_hbm, dst_v, buf_v, acc):
        cid = lax.axis_index("c")
        sid = lax.axis_index("s")
        wid = sid * NCORES + cid
        zv = jnp.zeros((16,), jnp.float32)
        ov = jnp.ones((16,), jnp.float32)

        def zrow(i, carry):
            buf_v[i] = zv
            return carry

        lax.fori_loop(0, CHUNK, zrow, 0)
        for k in range(ZFULL):
            pltpu.sync_copy(buf_v, acc.at[pl.ds(sid * ROWS_PER_SUB + k * CHUNK, CHUNK)])
        if ZREM:
            pltpu.sync_copy(buf_v.at[pl.ds(0, ZREM)],
                            acc.at[pl.ds(sid * ROWS_PER_SUB + ZFULL * CHUNK, ZREM)])
        pltpu.sync_copy(dst_hbm.at[wid], dst_v)
        plsc.subcore_barrier()

        def orow(i, carry):
            buf_v[i] = ov
            return carry

        lax.fori_loop(0, CHUNK, orow, 0)

        def chunk_body(g, carry):
            pltpu.sync_copy(buf_v, acc.at[dst_v.at[g]], add=True)
            return carry

        lax.fori_loop(0, nch, chunk_body, 0)
        plsc.subcore_barrier()
        pltpu.sync_copy(acc.at[pl.ds(sid * ROWS_PER_SUB, ROWS_PER_SUB)],
                        out_hbm.at[pl.ds(cid * NPAD + sid * ROWS_PER_SUB, ROWS_PER_SUB)])

    return kfn(dst3)


def _sc_gcn(tab, src3, dst3):
    n, co = tab.shape
    nch = src3.shape[1]
    ng = co // 16

    @functools.partial(
        pl.kernel,
        out_type=jax.ShapeDtypeStruct((NCORES * NPAD, co), jnp.float32),
        mesh=_sc_mesh(),
        compiler_params=_SC_PARAMS,
        scratch_types=[
            pltpu.VMEM((nch, CHUNK), jnp.int32),
            pltpu.VMEM((nch, CHUNK), jnp.int32),
            pltpu.VMEM((2, CHUNK, co), jnp.float32),
            pltpu.VMEM_SHARED((NPAD, co), jnp.float32),
            pltpu.SemaphoreType.DMA,
            pltpu.SemaphoreType.DMA,
        ],
    )
    def kfn(tab_hbm, src_hbm, dst_hbm, out_hbm, src_v, dst_v, buf_v, acc,
            sg0, sg1):
        cid = lax.axis_index("c")
        sid = lax.axis_index("s")
        wid = sid * NCORES + cid
        zv = jnp.zeros((16,), jnp.float32)
        sgs = (sg0, sg1)

        def zrow(i, carry):
            for j in range(ng):
                buf_v[0, i, pl.ds(16 * j, 16)] = zv
            return carry

        lax.fori_loop(0, CHUNK, zrow, 0)
        for k in range(ZFULL):
            pltpu.sync_copy(buf_v.at[0],
                            acc.at[pl.ds(sid * ROWS_PER_SUB + k * CHUNK, CHUNK)])
        if ZREM:
            pltpu.sync_copy(buf_v.at[0].at[pl.ds(0, ZREM)],
                            acc.at[pl.ds(sid * ROWS_PER_SUB + ZFULL * CHUNK, ZREM)])
        pltpu.sync_copy(src_hbm.at[wid], src_v)
        pltpu.sync_copy(dst_hbm.at[wid], dst_v)
        plsc.subcore_barrier()

        def fetch(g, slot):
            pltpu.async_copy(tab_hbm.at[src_v.at[g]], buf_v.at[slot], sgs[slot])

        def work(g, slot):
            pltpu.make_async_copy(tab_hbm.at[src_v.at[0]], buf_v.at[slot],
                                  sgs[slot]).wait()
            pltpu.sync_copy(buf_v.at[slot], acc.at[dst_v.at[g]], add=True)

        fetch(0, 0)
        if nch > 1:
            fetch(1, 1)

        def pair_body(k, carry):
            ga = 2 * k
            work(ga, 0)

            @pl.when(ga + 2 < nch)
            def _():
                fetch(ga + 2, 0)

            work(ga + 1, 1)

            @pl.when(ga + 3 < nch)
            def _():
                fetch(ga + 3, 1)

            return carry

        lax.fori_loop(0, nch // 2, pair_body, 0)
        if nch % 2:
            work(nch - 1, (nch - 1) % 2)

        plsc.subcore_barrier()
        pltpu.sync_copy(acc.at[pl.ds(sid * ROWS_PER_SUB, ROWS_PER_SUB)],
                        out_hbm.at[pl.ds(cid * NPAD + sid * ROWS_PER_SUB, ROWS_PER_SUB)])

    return kfn(tab, src3, dst3)


def _gen_layer(x, packs, src, dst, Ws, bs, W1, b1, W2, b2):
    splits = tuple(p[3] for p in packs)
    ns = len(splits)
    outs = _dense(x, Ws, bs, blk=2000, splits=splits)
    hs, hms = outs[:ns], outs[ns:]
    pairs = []
    for (ep, c0, em, w), h, hm in zip(packs, hs, hms):
        mb = (jnp.maximum(hm + em, 0.0) + EPS).reshape(w)
        acc = _sc_gen(h, ep, src, dst, mb, w, c0)
        pairs.append((acc, h))
    return _mlp(pairs, W1, b1, W2, b2)


def kernel(x, edge_index, edge_attr, g1_Ws, g1_bs, g1_We, g1_be, g1_W1,
           g1_b1, g1_W2, g1_b2, g2_Ws, g2_bs, g2_We, g2_be, g2_W1, g2_b1,
           g2_W2, g2_b2, mu_W, mu_b, ls_W, ls_b):
    ee = edge_index.shape[1]
    nch = ee // (NW * CHUNK)
    src = edge_index[0].reshape(NW, nch, CHUNK)
    dst = edge_index[1].reshape(NW, nch, CHUNK)

    dega = _sc_deg(dst)

    co1 = g1_Ws.shape[1]
    ha = co1 // 2
    co2 = g2_Ws.shape[1]
    hb = co2 // 2
    npad2 = 128 - (co1 - ha) - hb
    wp = jnp.concatenate(
        [g1_We, g2_We[:, :hb],
         jnp.pad(g2_We[:, hb:], ((0, 0), (0, 128 - (co2 - hb))))], axis=1)
    bp = jnp.concatenate(
        [g1_be, g2_be[:hb], jnp.pad(g2_be[hb:], (0, 128 - (co2 - hb)))])
    del npad2
    ep1, ep2, em1, em2 = _dense(edge_attr, wp, bp, blk=10000,
                                splits=(128, 128))
    l1_packs = [(ep1, 0, em1[:, :ha], ha),
                (ep1, ha, em1[:, ha:co1], co1 - ha)]
    l2_packs = [(ep1, co1, em1[:, co1:co1 + hb], hb),
                (ep2, 0, em2[:, :co2 - hb], co2 - hb)]

    x2 = _gen_layer(x, l1_packs, src, dst, g1_Ws, g1_bs,
                    g1_W1, g1_b1, g1_W2, g1_b2)
    x3 = _gen_layer(x2, l2_packs, src, dst, g2_Ws, g2_bs,
                    g2_W1, g2_b1, g2_W2, g2_b2)

    wcat = jnp.concatenate([mu_W, ls_W], axis=1)
    tab, slf, dv = _gcn_pre(x3, wcat, dega)
    gacc = _sc_gcn(tab, src, dst)
    return _gcn_post(gacc, slf, dv, mu_b, ls_b)


# speedup vs baseline: 20.4844x; 1.0001x over previous
import functools

import jax
import jax.numpy as jnp
from jax import lax
from jax.experimental import pallas as pl
from jax.experimental.pallas import tpu as pltpu
from jax.experimental.pallas import tpu_sc as plsc

EPS = 1e-7
NPAD = 10016
CHUNK = 80
NCORES = 2
NSUB = 16
NW = NCORES * NSUB
ROWS_PER_SUB = NPAD // NSUB
ZFULL = ROWS_PER_SUB // CHUNK
ZREM = ROWS_PER_SUB % CHUNK


def _dense_body(bounds, x_ref, w_ref, b_ref, *refs):
    ns = len(bounds) - 1
    i = pl.program_id(0)
    o = jnp.dot(x_ref[...], w_ref[...], preferred_element_type=jnp.float32)
    o = o + b_ref[...]
    for s in range(ns):
        lo, hi = bounds[s], bounds[s + 1]
        blkc = o[:, lo:hi]
        refs[s][...] = blkc
        bm = jnp.max(blkc, axis=0, keepdims=True)
        m_ref = refs[ns + s]

        @pl.when(i == 0)
        def _():
            m_ref[...] = bm

        @pl.when(i != 0)
        def _():
            m_ref[...] = jnp.maximum(m_ref[...], bm)


def _dense(x, w, b, blk, splits=None):
    n, k = x.shape
    co = w.shape[1]
    if splits is None:
        splits = (co,)
    bounds = [0]
    for s in splits:
        bounds.append(bounds[-1] + s)
    out_shape = [jax.ShapeDtypeStruct((n, s), jnp.float32) for s in splits]
    out_specs = [pl.BlockSpec((blk, s), lambda i: (i, 0)) for s in splits]
    out_shape += [jax.ShapeDtypeStruct((1, s), jnp.float32) for s in splits]
    out_specs += [pl.BlockSpec((1, s), lambda i: (0, 0)) for s in splits]
    return pl.pallas_call(
        functools.partial(_dense_body, tuple(bounds)),
        grid=(n // blk,),
        in_specs=[pl.BlockSpec((blk, k), lambda i: (i, 0)),
                  pl.BlockSpec((k, co), lambda i: (0, 0)),
                  pl.BlockSpec((1, co), lambda i: (0, 0))],
        out_specs=out_specs,
        out_shape=out_shape,
    )(x, w, b.reshape(1, co))


def _mlp_body(nh, w1_ref, b1_ref, w2_ref, b2_ref, *refs, o_ref):
    parts = []
    for s in range(nh):
        a0_ref, a1_ref, h_ref = refs[3 * s], refs[3 * s + 1], refs[3 * s + 2]
        ch = h_ref.shape[1]
        sm = a0_ref[:, :ch] + a1_ref[:, :ch]
        wm = a0_ref[:, ch:] + a1_ref[:, ch:]
        parts.append(wm / (sm + 1e-16) + h_ref[...])
    out = parts[0] if nh == 1 else jnp.concatenate(parts, axis=1)
    t = jnp.dot(out, w1_ref[...], preferred_element_type=jnp.float32) + b1_ref[...]
    t = jnp.maximum(t, 0.0)
    t = jnp.dot(t, w2_ref[...], preferred_element_type=jnp.float32) + b2_ref[...]
    o_ref[...] = jnp.maximum(t, 0.0)


def _mlp_wrap(nh, *args):
    o_ref = args[-1]
    w1_ref, b1_ref, w2_ref, b2_ref = args[:4]
    _mlp_body(nh, w1_ref, b1_ref, w2_ref, b2_ref, *args[4:-1], o_ref=o_ref)


def _mlp(pairs, w1, b1, w2, b2, blk=2000):
    co = w1.shape[0]
    ch_mlp = w1.shape[1]
    n = pairs[0][1].shape[0]
    ins = [w1, b1.reshape(1, ch_mlp), w2, b2.reshape(1, co)]
    in_specs = [pl.BlockSpec((co, ch_mlp), lambda i: (0, 0)),
                pl.BlockSpec((1, ch_mlp), lambda i: (0, 0)),
                pl.BlockSpec((ch_mlp, co), lambda i: (0, 0)),
                pl.BlockSpec((1, co), lambda i: (0, 0))]
    for acc, h in pairs:
        ch2 = acc.shape[1]
        ch = h.shape[1]
        ins += [acc[:NPAD], acc[NPAD:], h]
        in_specs += [pl.BlockSpec((blk, ch2), lambda i: (i, 0)),
                     pl.BlockSpec((blk, ch2), lambda i: (i, 0)),
                     pl.BlockSpec((blk, ch), lambda i: (i, 0))]
    return pl.pallas_call(
        functools.partial(_mlp_wrap, len(pairs)),
        grid=(n // blk,),
        in_specs=in_specs,
        out_specs=pl.BlockSpec((blk, co), lambda i: (i, 0)),
        out_shape=jax.ShapeDtypeStruct((n, co), jnp.float32),
    )(*ins)


def _gcnpre_body(x_ref, w_ref, d0_ref, d1_ref, tab_ref, slf_ref, dv_ref):
    deg = 1.0 + d0_ref[:, 0:1] + d1_ref[:, 0:1]
    dinv = lax.rsqrt(deg)
    hc = jnp.dot(x_ref[...], w_ref[...], preferred_element_type=jnp.float32)
    tab_ref[...] = hc * dinv
    slf_ref[...] = hc * (dinv * dinv)
    dv_ref[...] = dinv


def _gcn_pre(x, wcat, dega, blk=2000):
    n, k = x.shape
    co2 = wcat.shape[1]
    d0 = dega[:NPAD]
    d1 = dega[NPAD:]
    return pl.pallas_call(
        _gcnpre_body,
        grid=(n // blk,),
        in_specs=[pl.BlockSpec((blk, k), lambda i: (i, 0)),
                  pl.BlockSpec((k, co2), lambda i: (0, 0)),
                  pl.BlockSpec((blk, 16), lambda i: (i, 0)),
                  pl.BlockSpec((blk, 16), lambda i: (i, 0))],
        out_specs=[pl.BlockSpec((blk, co2), lambda i: (i, 0)),
                   pl.BlockSpec((blk, co2), lambda i: (i, 0)),
                   pl.BlockSpec((blk, 1), lambda i: (i, 0))],
        out_shape=[jax.ShapeDtypeStruct((n, co2), jnp.float32),
                   jax.ShapeDtypeStruct((n, co2), jnp.float32),
                   jax.ShapeDtypeStruct((n, 1), jnp.float32)],
    )(x, wcat, d0, d1)


def _gcnpost_body(co, g0_ref, g1_ref, slf_ref, dv_ref, bm_ref, bl_ref, mu_ref, ls_ref):
    agg = g0_ref[...] + g1_ref[...]
    out = dv_ref[...] * agg + slf_ref[...]
    mu_ref[...] = out[:, :co] + bm_ref[...]
    ls_ref[...] = out[:, co:2 * co] + bl_ref[...]


def _gcn_post(gacc, slf, dv, mu_b, ls_b, blk=2000):
    n = slf.shape[0]
    co2 = slf.shape[1]
    co = mu_b.shape[0]
    g0 = gacc[:NPAD]
    g1 = gacc[NPAD:]
    return pl.pallas_call(
        functools.partial(_gcnpost_body, co),
        grid=(n // blk,),
        in_specs=[pl.BlockSpec((blk, co2), lambda i: (i, 0)),
                  pl.BlockSpec((blk, co2), lambda i: (i, 0)),
                  pl.BlockSpec((blk, co2), lambda i: (i, 0)),
                  pl.BlockSpec((blk, 1), lambda i: (i, 0)),
                  pl.BlockSpec((1, co), lambda i: (0, 0)),
                  pl.BlockSpec((1, co), lambda i: (0, 0))],
        out_specs=[pl.BlockSpec((blk, co), lambda i: (i, 0)),
                   pl.BlockSpec((blk, co), lambda i: (i, 0))],
        out_shape=[jax.ShapeDtypeStruct((n, co), jnp.float32),
                   jax.ShapeDtypeStruct((n, co), jnp.float32)],
    )(g0, g1, slf, dv, mu_b.reshape(1, co), ls_b.reshape(1, co))


def _sc_mesh():
    return plsc.VectorSubcoreMesh(core_axis_name="c", subcore_axis_name="s")


_SC_PARAMS = pltpu.CompilerParams(use_tc_tiling_on_sc=False)


def _sc_gen(h, e, src3, dst3, mb, co, col0):
    n, cop = h.shape
    nch = src3.shape[1]
    per_tile = nch * CHUNK
    co2 = 2 * co
    ng = co // 16

    @functools.partial(
        pl.kernel,
        out_type=jax.ShapeDtypeStruct((NCORES * NPAD, co2), jnp.float32),
        mesh=_sc_mesh(),
        compiler_params=_SC_PARAMS,
        scratch_types=[
            pltpu.VMEM((nch, CHUNK), jnp.int32),
            pltpu.VMEM((nch, CHUNK), jnp.int32),
            pltpu.VMEM((2, CHUNK, cop), jnp.float32),
            pltpu.VMEM((2, CHUNK, cop), jnp.float32),
            pltpu.VMEM((2, CHUNK, co2), jnp.float32),
            pltpu.VMEM((cop,), jnp.float32),
            pltpu.VMEM_SHARED((NPAD, co2), jnp.float32),
            pltpu.SemaphoreType.DMA,
            pltpu.SemaphoreType.DMA,
            pltpu.SemaphoreType.DMA,
            pltpu.SemaphoreType.DMA,
            pltpu.SemaphoreType.DMA,
            pltpu.SemaphoreType.DMA,
        ],
    )
    def kfn(h_hbm, e_hbm, src_hbm, dst_hbm, mb_hbm, out_hbm,
            src_v, dst_v, h_v, e_v, o_v, mb_v, acc,
            sh0, sh1, se0, se1, ss0, ss1):
        cid = lax.axis_index("c")
        sid = lax.axis_index("s")
        wid = sid * NCORES + cid
        zv = jnp.zeros((16,), jnp.float32)
        shs = (sh0, sh1)
        ses = (se0, se1)
        sss = (ss0, ss1)

        def zrow(i, carry):
            for j in range(co2 // 16):
                o_v[0, i, pl.ds(16 * j, 16)] = zv
            return carry

        lax.fori_loop(0, CHUNK, zrow, 0)
        for k in range(ZFULL):
            pltpu.sync_copy(o_v.at[0],
                            acc.at[pl.ds(sid * ROWS_PER_SUB + k * CHUNK, CHUNK)])
        if ZREM:
            pltpu.sync_copy(o_v.at[0].at[pl.ds(0, ZREM)],
                            acc.at[pl.ds(sid * ROWS_PER_SUB + ZFULL * CHUNK, ZREM)])

        pltpu.sync_copy(mb_hbm, mb_v)
        pltpu.sync_copy(src_hbm.at[wid], src_v)
        pltpu.sync_copy(dst_hbm.at[wid], dst_v)
        mbs = [mb_v[pl.ds(16 * j, 16)] for j in range(ng)]
        plsc.subcore_barrier()

        def fetch(g, slot):
            base = wid * per_tile + g * CHUNK
            pltpu.async_copy(h_hbm.at[src_v.at[g]], h_v.at[slot], shs[slot])
            pltpu.async_copy(e_hbm.at[pl.ds(base, CHUNK), pl.ds(col0, cop)],
                             e_v.at[slot], ses[slot])

        def wait(slot):
            pltpu.make_async_copy(h_hbm.at[src_v.at[0]], h_v.at[slot], shs[slot]).wait()
            pltpu.make_async_copy(e_hbm.at[pl.ds(0, CHUNK), pl.ds(col0, cop)],
                                  e_v.at[slot], ses[slot]).wait()

        def wait_scat(slot):
            pltpu.make_async_copy(o_v.at[slot], acc.at[dst_v.at[0]], sss[slot]).wait()

        def work(g, slot, first):
            wait(slot)
            if not first:
                wait_scat(slot)

            @plsc.parallel_loop(0, CHUNK, unroll=8)
            def edge_body(i):
                for j in range(ng):
                    m = (h_v[slot, i, pl.ds(16 * j, 16)]
                         + e_v[slot, i, pl.ds(16 * j, 16)])
                    m = jnp.maximum(m, 0.0) + EPS
                    t = jnp.exp(m - mbs[j])
                    o_v[slot, i, pl.ds(16 * j, 16)] = t
                    o_v[slot, i, pl.ds(co + 16 * j, 16)] = m * t

            pltpu.async_copy(o_v.at[slot], acc.at[dst_v.at[g]], sss[slot],
                             add=True)

        fetch(0, 0)
        if nch > 1:
            fetch(1, 1)
        work(0, 0, True)
        if nch > 2:
            fetch(2, 0)
        work(1, 1, True)
        if nch > 3:
            fetch(3, 1)

        def pair_body(k, carry):
            ga = 2 * k + 2
            work(ga, 0, False)

            @pl.when(ga + 2 < nch)
            def _():
                fetch(ga + 2, 0)

            work(ga + 1, 1, False)

            @pl.when(ga + 3 < nch)
            def _():
                fetch(ga + 3, 1)

            return carry

        lax.fori_loop(0, (nch - 2) // 2, pair_body, 0)
        if nch % 2:
            work(nch - 1, (nch - 1) % 2, False)
        wait_scat(0)
        wait_scat(1)

        plsc.subcore_barrier()
        pltpu.sync_copy(acc.at[pl.ds(sid * ROWS_PER_SUB, ROWS_PER_SUB)],
                        out_hbm.at[pl.ds(cid * NPAD + sid * ROWS_PER_SUB, ROWS_PER_SUB)])

    return kfn(h, e, src3, dst3, mb)


def _sc_deg(dst3):
    nch = dst3.shape[1]

    @functools.partial(
        pl.kernel,
        out_type=jax.ShapeDtypeStruct((NCORES * NPAD, 16), jnp.float32),
        mesh=_sc_mesh(),
        compiler_params=_SC_PARAMS,
        scratch_types=[
            pltpu.VMEM((nch, CHUNK), jnp.int32),
            pltpu.VMEM((CHUNK, 16), jnp.float32),
            pltpu.VMEM_SHARED((NPAD, 16), jnp.float32),
        ],
    )
    def kfn(dst_hbm, out_hbm, dst_v, buf_v, acc):
        cid = lax.axis_index("c")
        sid = lax.axis_index("s")
        wid = sid * NCORES + cid
        zv = jnp.zeros((16,), jnp.float32)
        ov = jnp.ones((16,), jnp.float32)

        def zrow(i, carry):
            buf_v[i] = zv
            return carry

        lax.fori_loop(0, CHUNK, zrow, 0)
        for k in range(ZFULL):
            pltpu.sync_copy(buf_v, acc.at[pl.ds(sid * ROWS_PER_SUB + k * CHUNK, CHUNK)])
        if ZREM:
            pltpu.sync_copy(buf_v.at[pl.ds(0, ZREM)],
                            acc.at[pl.ds(sid * ROWS_PER_SUB + ZFULL * CHUNK, ZREM)])
        pltpu.sync_copy(dst_hbm.at[wid], dst_v)
        plsc.subcore_barrier()

        def orow(i, carry):
            buf_v[i] = ov
            return carry

        lax.fori_loop(0, CHUNK, orow, 0)

        def chunk_body(g, carry):
            pltpu.sync_copy(buf_v, acc.at[dst_v.at[g]], add=True)
            return carry

        lax.fori_loop(0, nch, chunk_body, 0)
        plsc.subcore_barrier()
        pltpu.sync_copy(acc.at[pl.ds(sid * ROWS_PER_SUB, ROWS_PER_SUB)],
                        out_hbm.at[pl.ds(cid * NPAD + sid * ROWS_PER_SUB, ROWS_PER_SUB)])

    return kfn(dst3)


def _sc_gcn(tab, src3, dst3):
    n, co = tab.shape
    nch = src3.shape[1]
    ng = co // 16

    @functools.partial(
        pl.kernel,
        out_type=jax.ShapeDtypeStruct((NCORES * NPAD, co), jnp.float32),
        mesh=_sc_mesh(),
        compiler_params=_SC_PARAMS,
        scratch_types=[
            pltpu.VMEM((nch, CHUNK), jnp.int32),
            pltpu.VMEM((nch, CHUNK), jnp.int32),
            pltpu.VMEM((2, CHUNK, co), jnp.float32),
            pltpu.VMEM_SHARED((NPAD, co), jnp.float32),
            pltpu.SemaphoreType.DMA,
            pltpu.SemaphoreType.DMA,
        ],
    )
    def kfn(tab_hbm, src_hbm, dst_hbm, out_hbm, src_v, dst_v, buf_v, acc,
            sg0, sg1):
        cid = lax.axis_index("c")
        sid = lax.axis_index("s")
        wid = sid * NCORES + cid
        zv = jnp.zeros((16,), jnp.float32)
        sgs = (sg0, sg1)

        def zrow(i, carry):
            for j in range(ng):
                buf_v[0, i, pl.ds(16 * j, 16)] = zv
            return carry

        lax.fori_loop(0, CHUNK, zrow, 0)
        for k in range(ZFULL):
            pltpu.sync_copy(buf_v.at[0],
                            acc.at[pl.ds(sid * ROWS_PER_SUB + k * CHUNK, CHUNK)])
        if ZREM:
            pltpu.sync_copy(buf_v.at[0].at[pl.ds(0, ZREM)],
                            acc.at[pl.ds(sid * ROWS_PER_SUB + ZFULL * CHUNK, ZREM)])
        pltpu.sync_copy(src_hbm.at[wid], src_v)
        pltpu.sync_copy(dst_hbm.at[wid], dst_v)
        plsc.subcore_barrier()

        def fetch(g, slot):
            pltpu.async_copy(tab_hbm.at[src_v.at[g]], buf_v.at[slot], sgs[slot])

        def work(g, slot):
            pltpu.make_async_copy(tab_hbm.at[src_v.at[0]], buf_v.at[slot],
                                  sgs[slot]).wait()
            pltpu.sync_copy(buf_v.at[slot], acc.at[dst_v.at[g]], add=True)

        fetch(0, 0)
        if nch > 1:
            fetch(1, 1)

        def pair_body(k, carry):
            ga = 2 * k
            work(ga, 0)

            @pl.when(ga + 2 < nch)
            def _():
                fetch(ga + 2, 0)

            work(ga + 1, 1)

            @pl.when(ga + 3 < nch)
            def _():
                fetch(ga + 3, 1)

            return carry

        lax.fori_loop(0, nch // 2, pair_body, 0)
        if nch % 2:
            work(nch - 1, (nch - 1) % 2)

        plsc.subcore_barrier()
        pltpu.sync_copy(acc.at[pl.ds(sid * ROWS_PER_SUB, ROWS_PER_SUB)],
                        out_hbm.at[pl.ds(cid * NPAD + sid * ROWS_PER_SUB, ROWS_PER_SUB)])

    return kfn(tab, src3, dst3)


def _gen_layer(x, packs, src, dst, Ws, bs, W1, b1, W2, b2):
    splits = tuple(p[3] for p in packs)
    ns = len(splits)
    outs = _dense(x, Ws, bs, blk=2000, splits=splits)
    hs, hms = outs[:ns], outs[ns:]
    pairs = []
    for (ep, c0, em, w), h, hm in zip(packs, hs, hms):
        mb = (jnp.maximum(hm + em, 0.0) + EPS).reshape(w)
        acc = _sc_gen(h, ep, src, dst, mb, w, c0)
        pairs.append((acc, h))
    return _mlp(pairs, W1, b1, W2, b2)


def kernel(x, edge_index, edge_attr, g1_Ws, g1_bs, g1_We, g1_be, g1_W1,
           g1_b1, g1_W2, g1_b2, g2_Ws, g2_bs, g2_We, g2_be, g2_W1, g2_b1,
           g2_W2, g2_b2, mu_W, mu_b, ls_W, ls_b):
    ee = edge_index.shape[1]
    nch = ee // (NW * CHUNK)
    src = edge_index[0].reshape(NW, nch, CHUNK)
    dst = edge_index[1].reshape(NW, nch, CHUNK)

    dega = _sc_deg(dst)

    co1 = g1_Ws.shape[1]
    ha = co1 // 2
    co2 = g2_Ws.shape[1]
    hb = co2 // 2
    wp = jnp.concatenate(
        [g1_We, g2_We[:, :hb],
         jnp.pad(g2_We[:, hb:], ((0, 0), (0, 128 - (co2 - hb))))], axis=1)
    bp = jnp.concatenate(
        [g1_be, g2_be[:hb], jnp.pad(g2_be[hb:], (0, 128 - (co2 - hb)))])
    ep1, ep2, em1, em2 = _dense(edge_attr, wp, bp, blk=10000,
                                splits=(128, 128))
    l1_packs = [(ep1, 0, em1[:, :ha], ha),
                (ep1, ha, em1[:, ha:co1], co1 - ha)]
    l2_packs = [(ep1, co1, em1[:, co1:co1 + hb], hb),
                (ep2, 0, em2[:, :co2 - hb], co2 - hb)]

    x2 = _gen_layer(x, l1_packs, src, dst, g1_Ws, g1_bs,
                    g1_W1, g1_b1, g1_W2, g1_b2)
    x3 = _gen_layer(x2, l2_packs, src, dst, g2_Ws, g2_bs,
                    g2_W1, g2_b1, g2_W2, g2_b2)

    wcat = jnp.concatenate([mu_W, ls_W], axis=1)
    tab, slf, dv = _gcn_pre(x3, wcat, dega)
    gacc = _sc_gcn(tab, src, dst)
    return _gcn_post(gacc, slf, dv, mu_b, ls_b)
